# baseline (speedup 1.0000x reference)
import functools

import jax
import jax.numpy as jnp
from jax import lax
from jax.experimental import pallas as pl
from jax.experimental.pallas import tpu as pltpu
from jax.experimental.pallas import tpu_sc as plsc

_NC = 2
_NS = 16
_NW = _NC * _NS


def _softplus(x):
    return jnp.maximum(x, 0.0) + jnp.log1p(jnp.exp(-jnp.abs(x)))


def _bd4(w):
    a, b = w.shape
    z = jnp.zeros((a, b), w.dtype)
    rows = [jnp.concatenate([w if i == j else z for j in range(4)], axis=1)
            for i in range(4)]
    return jnp.concatenate(rows, axis=0)


def _gather_body(n, nblk, blk, atoms_hbm, idx1_hbm, idx2_hbm,
                 out1_hbm, out2_hbm, idx_v0, idx_v1, rows_v0, rows_v1,
                 atoms_sp, semi, semg, semw):
    c = lax.axis_index("c")
    s = lax.axis_index("s")
    wid = s * _NC + c
    ew = nblk * blk
    base = wid * ew

    chunk = (n // _NS) // 8 * 8
    rem = n - chunk * _NS
    pltpu.sync_copy(atoms_hbm.at[pl.ds(s * chunk, chunk)],
                    rows_v0.at[pl.ds(0, chunk)])
    pltpu.sync_copy(rows_v0.at[pl.ds(0, chunk)],
                    atoms_sp.at[pl.ds(s * chunk, chunk)])
    if rem:
        @pl.when(s == 0)
        def _():
            pltpu.sync_copy(atoms_hbm.at[pl.ds(chunk * _NS, rem)],
                            rows_v0.at[pl.ds(0, rem)])
            pltpu.sync_copy(rows_v0.at[pl.ds(0, rem)],
                            atoms_sp.at[pl.ds(chunk * _NS, rem)])
    plsc.subcore_barrier()

    idx_bufs = (idx_v0, idx_v1)
    row_bufs = (rows_v0, rows_v1)
    tasks = ([(idx1_hbm, out1_hbm, t) for t in range(nblk)]
             + [(idx2_hbm, out2_hbm, t) for t in range(nblk)])
    nt = len(tasks)
    w_d = [None, None]

    ih0, _, t0 = tasks[0]
    i_d = [pltpu.async_copy(ih0.at[pl.ds(base + t0 * blk, blk)],
                            idx_bufs[0], semi), None]
    for i, (ih, oh, t) in enumerate(tasks):
        b = i % 2
        nb = (i + 1) % 2
        if i + 1 < nt:
            ihn, _, tn = tasks[i + 1]
            i_d[nb] = pltpu.async_copy(
                ihn.at[pl.ds(base + tn * blk, blk)], idx_bufs[nb], semi)
        i_d[b].wait()
        if w_d[b] is not None:
            w_d[b].wait()
        pltpu.async_copy(atoms_sp.at[idx_bufs[b]], row_bufs[b], semg).wait()
        w_d[b] = pltpu.async_copy(row_bufs[b],
                                  oh.at[pl.ds(base + t * blk, blk)], semw)
    for d in w_d:
        if d is not None:
            d.wait()


def _sc_gather(atoms2, idx1, idx2):
    n, d = atoms2.shape
    e = idx1.shape[0]
    ew = e // _NW
    assert ew * _NW == e
    blk = 1000
    nblk = ew // blk
    assert nblk * blk == ew
    mesh = plsc.VectorSubcoreMesh(core_axis_name="c", subcore_axis_name="s")
    stage = max(blk, (n // _NS) // 8 * 8 + (n - (n // _NS) // 8 * 8 * _NS))
    f = pl.kernel(
        functools.partial(_gather_body, n, nblk, blk),
        out_type=(
            jax.ShapeDtypeStruct((e, d), jnp.float32),
            jax.ShapeDtypeStruct((e, d), jnp.float32),
        ),
        mesh=mesh,
        scratch_types=[
            pltpu.VMEM((blk,), jnp.int32),
            pltpu.VMEM((blk,), jnp.int32),
            pltpu.VMEM((stage, d), jnp.float32),
            pltpu.VMEM((stage, d), jnp.float32),
            pltpu.VMEM_SHARED((n, d), jnp.float32),
            pltpu.SemaphoreType.DMA,
            pltpu.SemaphoreType.DMA,
            pltpu.SemaphoreType.DMA,
        ],
        compiler_params=pltpu.CompilerParams(use_tc_tiling_on_sc=False),
    )
    return f(atoms2, idx1, idx2)


def _scatter_body(n, nblk, blk, zeros32_hbm, zeros16_hbm, ones_hbm,
                  idx_hbm, vals_hbm, sums_out, counts_out,
                  idx_v0, idx_v1, vals_v0, vals_v1, ones_v, cz_v,
                  sums_sp, counts_sp, semi, semv, sems):
    c = lax.axis_index("c")
    s = lax.axis_index("s")
    chunk = (n // _NS) // 8 * 8
    rem = n - chunk * _NS
    ew = nblk * blk
    base = (c * _NS + s) * ew

    pltpu.sync_copy(ones_hbm, ones_v)

    def zero_slice(off, ln):
        pltpu.sync_copy(zeros32_hbm.at[pl.ds(off, ln)],
                        vals_v0.at[pl.ds(0, ln)])
        pltpu.sync_copy(vals_v0.at[pl.ds(0, ln)], sums_sp.at[pl.ds(off, ln)])
        pltpu.sync_copy(zeros16_hbm.at[pl.ds(off, ln)],
                        cz_v.at[pl.ds(0, ln)])
        pltpu.sync_copy(cz_v.at[pl.ds(0, ln)],
                        counts_sp.at[pl.ds(off, ln)])

    zero_slice(s * chunk, chunk)
    if rem:
        @pl.when(s == 0)
        def _():
            zero_slice(chunk * _NS, rem)
    plsc.subcore_barrier()

    idx_bufs = (idx_v0, idx_v1)
    val_bufs = (vals_v0, vals_v1)
    s_d = [None, None]

    i_d = [(pltpu.async_copy(idx_hbm.at[pl.ds(base, blk)], idx_v0, semi),
            pltpu.async_copy(vals_hbm.at[pl.ds(base, blk)], vals_v0, semv)),
           None]
    for t in range(nblk):
        b = t % 2
        nb = (t + 1) % 2
        if t + 1 < nblk:
            if s_d[nb] is not None:
                for dsc in s_d[nb]:
                    dsc.wait()
                s_d[nb] = None
            off = base + (t + 1) * blk
            i_d[nb] = (
                pltpu.async_copy(idx_hbm.at[pl.ds(off, blk)],
                                 idx_bufs[nb], semi),
                pltpu.async_copy(vals_hbm.at[pl.ds(off, blk)],
                                 val_bufs[nb], semv))
        i_d[b][0].wait()
        i_d[b][1].wait()
        subs = []
        off_s = 0
        while off_s < blk:
            ln = min(128, blk - off_s)
            subs.append(pltpu.async_copy(
                val_bufs[b].at[pl.ds(off_s, ln)],
                sums_sp.at[idx_bufs[b].at[pl.ds(off_s, ln)]], sems,
                add=True))
            subs.append(pltpu.async_copy(
                ones_v.at[pl.ds(0, ln)],
                counts_sp.at[idx_bufs[b].at[pl.ds(off_s, ln)]], sems,
                add=True))
            off_s += ln
        s_d[b] = subs
    for dd in s_d:
        if dd is not None:
            for dsc in dd:
                dsc.wait()

    plsc.subcore_barrier()

    def out_slice(off, ln):
        pltpu.sync_copy(sums_sp.at[pl.ds(off, ln)], vals_v0.at[pl.ds(0, ln)])
        pltpu.sync_copy(vals_v0.at[pl.ds(0, ln)],
                        sums_out.at[c].at[pl.ds(off, ln)])
        pltpu.sync_copy(counts_sp.at[pl.ds(off, ln)],
                        cz_v.at[pl.ds(0, ln)])
        pltpu.sync_copy(cz_v.at[pl.ds(0, ln)],
                        counts_out.at[c].at[pl.ds(off, ln)])

    out_slice(s * chunk, chunk)
    if rem:
        @pl.when(s == 0)
        def _():
            out_slice(chunk * _NS, rem)


def _sc_scatter(n, idx1, vals):
    e = vals.shape[0]
    d = vals.shape[1]
    ew = e // _NW
    assert ew * _NW == e
    blk = 1000
    nblk = ew // blk
    assert nblk * blk == ew
    chunk = (n // _NS) // 8 * 8
    rem = n - chunk * _NS
    stage = max(blk, chunk + rem)
    mesh = plsc.VectorSubcoreMesh(core_axis_name="c", subcore_axis_name="s")
    f = pl.kernel(
        functools.partial(_scatter_body, n, nblk, blk),
        out_type=(
            jax.ShapeDtypeStruct((_NC, n, d), jnp.float32),
            jax.ShapeDtypeStruct((_NC, n, 16), jnp.float32),
        ),
        mesh=mesh,
        scratch_types=[
            pltpu.VMEM((blk,), jnp.int32),
            pltpu.VMEM((blk,), jnp.int32),
            pltpu.VMEM((stage, d), jnp.float32),
            pltpu.VMEM((stage, d), jnp.float32),
            pltpu.VMEM((blk, 16), jnp.float32),
            pltpu.VMEM((stage, 16), jnp.float32),
            pltpu.VMEM_SHARED((n, d), jnp.float32),
            pltpu.VMEM_SHARED((n, 16), jnp.float32),
            pltpu.SemaphoreType.DMA,
            pltpu.SemaphoreType.DMA,
            pltpu.SemaphoreType.DMA,
        ],
        compiler_params=pltpu.CompilerParams(use_tc_tiling_on_sc=False),
    )
    zeros32 = jnp.zeros((n, d), jnp.float32)
    zeros16 = jnp.zeros((n, 16), jnp.float32)
    ones = jnp.ones((blk, 16), jnp.float32)
    return f(zeros32, zeros16, ones, idx1, vals)


def _edge_body(g1_ref, g2_ref, bt_ref, st_ref, sel_ref,
               w1a_ref, w1b_ref, w1c_ref, w1d_ref, b1_ref,
               w2_ref, b2_ref, w3_ref, b3_ref,
               out_ref, bsum_ref):
    i = pl.program_id(0)
    bf = jnp.bfloat16
    stt = jnp.dot(st_ref[...], w1d_ref[...],
                  preferred_element_type=jnp.float32) + b1_ref[...]
    stt4 = jnp.concatenate([stt, stt, stt, stt], axis=1)

    sel = sel_ref[...].astype(bf)
    b_rows = []
    for cchunk in range(4):
        btc = bt_ref[:, cchunk * 800:(cchunk + 1) * 800].astype(bf)
        parts = [
            lax.dot_general(sel[j], btc, (((0,), (1,)), ((), ())),
                            preferred_element_type=jnp.float32)
            for j in range(4)
        ]
        b_rows.append(jnp.concatenate(parts, axis=1))
    b_packed = jnp.concatenate(b_rows, axis=0)

    x = (jnp.dot(g1_ref[...].astype(bf), w1a_ref[...],
                 preferred_element_type=jnp.float32)
         + jnp.dot(g2_ref[...].astype(bf), w1b_ref[...],
                   preferred_element_type=jnp.float32)
         + jnp.dot(b_packed.astype(bf), w1c_ref[...],
                   preferred_element_type=jnp.float32)
         + stt4)
    h = _softplus(x)
    h = _softplus(jnp.dot(h.astype(bf), w2_ref[...],
                          preferred_element_type=jnp.float32) + b2_ref[...])
    o = _softplus(jnp.dot(h.astype(bf), w3_ref[...],
                          preferred_element_type=jnp.float32) + b3_ref[...])
    out_ref[...] = o

    @pl.when(i == 0)
    def _():
        bsum_ref[...] = jnp.zeros_like(bsum_ref)

    bsum_ref[0:1, :] += jnp.sum(o, axis=0, keepdims=True)


def _tc_edge_mlp(g1p, g2p, bt, st_row, w1a, w1b, w1c, w1d,
                 b1, w2, b2, w3, b3):
    e4 = g1p.shape[0]
    bf = jnp.bfloat16
    sel = (jnp.arange(800)[None, :, None]
           == 4 * jnp.arange(200)[None, None, :]
           + jnp.arange(4)[:, None, None]).astype(jnp.float32)
    w1a_bd = _bd4(w1a).astype(bf)
    w1b_bd = _bd4(w1b).astype(bf)
    w1c_bd = _bd4(w1c).astype(bf)
    b2t = jnp.tile(b2, 4)[None, :]
    w2_bd = _bd4(w2).astype(bf)
    w3_bd = _bd4(w3).astype(bf)
    b3t = jnp.tile(b3, 4)[None, :]
    r = 800
    grid = e4 // r
    assert grid * r == e4
    row_spec = pl.BlockSpec((r, 128), lambda i: (i, 0))
    bt_spec = pl.BlockSpec((32, 4 * r), lambda i: (0, i))

    def fs(x):
        return pl.BlockSpec(x.shape, lambda i: tuple(0 for _ in x.shape))

    args = (g1p, g2p, bt, st_row, sel, w1a_bd, w1b_bd, w1c_bd,
            w1d, b1[None, :], w2_bd, b2t, w3_bd, b3t)
    out, bsum = pl.pallas_call(
        _edge_body,
        grid=(grid,),
        in_specs=[row_spec, row_spec, bt_spec] + [fs(a) for a in args[3:]],
        out_specs=[pl.BlockSpec((r, 128), lambda i: (i, 0)),
                   pl.BlockSpec((8, 128), lambda i: (0, 0))],
        out_shape=[jax.ShapeDtypeStruct((e4, 128), jnp.float32),
                   jax.ShapeDtypeStruct((8, 128), jnp.float32)],
    )(*args)
    return out, bsum


def _node_body(e_edges, n_atoms, sums_ref, counts_ref,
               atoms_ref, st_ref, bsum_ref, fold_ref,
               wv1a_ref, wv1b_ref, wv1c_ref, bv1_ref, wv2_ref, bv2_ref,
               wv3_ref, bv3_ref,
               wu1a_ref, wu1b_ref, wu1c_ref, bu1_ref, wu2_ref, bu2_ref,
               wu3_ref, bu3_ref,
               atoms_out_ref, state_out_ref):
    bf = jnp.bfloat16
    ssum = sums_ref[0] + sums_ref[1]
    cnt = counts_ref[...]
    bta = ssum / cnt
    st = st_ref[...]
    stt = (jnp.dot(st, wv1c_ref[...], preferred_element_type=jnp.float32)
           + bv1_ref[...])
    stt4 = jnp.concatenate([stt, stt, stt, stt], axis=1)
    x = (jnp.dot(bta.astype(bf), wv1a_ref[...],
                 preferred_element_type=jnp.float32)
         + jnp.dot(atoms_ref[...].astype(bf), wv1b_ref[...],
                   preferred_element_type=jnp.float32)
         + stt4)
    h = _softplus(x)
    h = _softplus(jnp.dot(h.astype(bf), wv2_ref[...],
                          preferred_element_type=jnp.float32) + bv2_ref[...])
    atoms_new = _softplus(
        jnp.dot(h.astype(bf), wv3_ref[...],
                preferred_element_type=jnp.float32) + bv3_ref[...])
    atoms_out_ref[...] = atoms_new

    fold = fold_ref[...]
    asum = jnp.dot(jnp.sum(atoms_new, axis=0, keepdims=True), fold,
                   preferred_element_type=jnp.float32)
    bsum = jnp.dot(jnp.sum(bsum_ref[...], axis=0, keepdims=True),
                   fold, preferred_element_type=jnp.float32)
    bts = bsum * (1.0 / e_edges)
    ats = asum * (1.0 / n_atoms)
    xs = (jnp.dot(bts, wu1a_ref[...], preferred_element_type=jnp.float32)
          + jnp.dot(ats, wu1b_ref[...], preferred_element_type=jnp.float32)
          + jnp.dot(st, wu1c_ref[...], preferred_element_type=jnp.float32)
          + bu1_ref[...])
    hs = _softplus(xs)
    hs = _softplus(jnp.dot(hs, wu2_ref[...],
                           preferred_element_type=jnp.float32) + bu2_ref[...])
    sn = _softplus(jnp.dot(hs, wu3_ref[...],
                           preferred_element_type=jnp.float32) + bu3_ref[...])
    state_out_ref[...] = jnp.broadcast_to(sn, state_out_ref.shape)


def _tc_node_state(e_edges, n_atoms, sums_p, counts_p, atoms_p,
                   st_row, bsum, params):
    n4 = atoms_p.shape[0]
    fold = jnp.tile(jnp.eye(32, dtype=jnp.float32), (4, 1))

    def fs(x):
        return pl.BlockSpec(x.shape, lambda: tuple(0 for _ in x.shape))

    args = (sums_p, counts_p, atoms_p, st_row, bsum, fold) + tuple(params)
    atoms_new, state_new = pl.pallas_call(
        functools.partial(_node_body, e_edges, n_atoms),
        grid=(),
        in_specs=[fs(a) for a in args],
        out_specs=[pl.BlockSpec((n4, 128), lambda: (0, 0)),
                   pl.BlockSpec((8, 32), lambda: (0, 0))],
        out_shape=[jax.ShapeDtypeStruct((n4, 128), jnp.float32),
                   jax.ShapeDtypeStruct((8, 32), jnp.float32)],
    )(*args)
    return atoms_new, state_new


def kernel(bonds, bond_atom_1, bond_atom_2, atoms, state,
           W_e1, b_e1, W_e2, b_e2, W_e3, b_e3,
           W_v1, b_v1, W_v2, b_v2, W_v3, b_v3,
           W_u1, b_u1, W_u2, b_u2, W_u3, b_u3):
    b, e, d = bonds.shape
    n = atoms.shape[1]
    e4, n4 = e // 4, n // 4
    bonds2 = bonds.reshape(e, d)
    atoms2 = atoms.reshape(n, d)
    idx1 = bond_atom_1.reshape(e).astype(jnp.int32)
    idx2 = bond_atom_2.reshape(e).astype(jnp.int32)
    st_row = state.reshape(1, d)

    bt = jnp.swapaxes(bonds, 1, 2).reshape(d, e)
    g1, g2 = _sc_gather(atoms2, idx1, idx2)
    bn_p, bsum = _tc_edge_mlp(
        g1.reshape(e4, 128), g2.reshape(e4, 128), bt, st_row,
        W_e1[0:32], W_e1[32:64], W_e1[64:96], W_e1[96:128], b_e1,
        W_e2, b_e2, W_e3, b_e3)

    sums, counts = _sc_scatter(n, idx1, bn_p.reshape(e, 32))

    counts16 = counts[0] + counts[1]
    counts_p = jnp.tile(counts16, (1, 2)).reshape(n4, 128)

    bf = jnp.bfloat16
    node_params = (
        _bd4(W_v1[0:32]).astype(bf), _bd4(W_v1[32:64]).astype(bf),
        W_v1[64:96], b_v1.reshape(1, -1),
        _bd4(W_v2).astype(bf), jnp.tile(b_v2, 4)[None, :],
        _bd4(W_v3).astype(bf), jnp.tile(b_v3, 4)[None, :],
        W_u1[0:32], W_u1[32:64], W_u1[64:96], b_u1.reshape(1, -1),
        W_u2, b_u2.reshape(1, -1), W_u3, b_u3.reshape(1, -1))
    atoms_new, state_new = _tc_node_state(
        float(e), float(n), sums.reshape(2, n4, 128), counts_p,
        atoms2.reshape(n4, 128), st_row, bsum, node_params)

    return (bn_p.reshape(b, e, 32),
            atoms_new.reshape(b, n, 32),
            state_new[0:1, :].reshape(b, 1, 32))


# speedup vs baseline: 1.1467x; 1.1467x over previous
import functools

import jax
import jax.numpy as jnp
from jax import lax
from jax.experimental import pallas as pl
from jax.experimental.pallas import tpu as pltpu
from jax.experimental.pallas import tpu_sc as plsc

_NC = 2
_NS = 16
_NW = _NC * _NS


def _softplus(x):
    return jnp.maximum(x, 0.0) + jnp.log1p(jnp.exp(-jnp.abs(x)))


def _bd4(w):
    a, b = w.shape
    z = jnp.zeros((a, b), w.dtype)
    rows = [jnp.concatenate([w if i == j else z for j in range(4)], axis=1)
            for i in range(4)]
    return jnp.concatenate(rows, axis=0)


def _gather_body(n, nblk, blk, atoms_hbm, idx1_hbm, idx2_hbm,
                 out1_hbm, out2_hbm, idx_v0, idx_v1, rows_v0, rows_v1,
                 atoms_sp, semi, semg, semw):
    c = lax.axis_index("c")
    s = lax.axis_index("s")
    wid = s * _NC + c
    ew = nblk * blk
    base = wid * ew

    chunk = (n // _NS) // 8 * 8
    rem = n - chunk * _NS
    pltpu.sync_copy(atoms_hbm.at[pl.ds(s * chunk, chunk)],
                    rows_v0.at[pl.ds(0, chunk)])
    pltpu.sync_copy(rows_v0.at[pl.ds(0, chunk)],
                    atoms_sp.at[pl.ds(s * chunk, chunk)])
    if rem:
        @pl.when(s == 0)
        def _():
            pltpu.sync_copy(atoms_hbm.at[pl.ds(chunk * _NS, rem)],
                            rows_v0.at[pl.ds(0, rem)])
            pltpu.sync_copy(rows_v0.at[pl.ds(0, rem)],
                            atoms_sp.at[pl.ds(chunk * _NS, rem)])
    plsc.subcore_barrier()

    idx_bufs = (idx_v0, idx_v1)
    row_bufs = (rows_v0, rows_v1)
    tasks = ([(idx1_hbm, out1_hbm, t) for t in range(nblk)]
             + [(idx2_hbm, out2_hbm, t) for t in range(nblk)])
    nt = len(tasks)
    w_d = [None, None]

    ih0, _, t0 = tasks[0]
    i_d = [pltpu.async_copy(ih0.at[pl.ds(base + t0 * blk, blk)],
                            idx_bufs[0], semi), None]
    for i, (ih, oh, t) in enumerate(tasks):
        b = i % 2
        nb = (i + 1) % 2
        if i + 1 < nt:
            ihn, _, tn = tasks[i + 1]
            i_d[nb] = pltpu.async_copy(
                ihn.at[pl.ds(base + tn * blk, blk)], idx_bufs[nb], semi)
        i_d[b].wait()
        if w_d[b] is not None:
            w_d[b].wait()
        pltpu.async_copy(atoms_sp.at[idx_bufs[b]], row_bufs[b], semg).wait()
        w_d[b] = pltpu.async_copy(row_bufs[b],
                                  oh.at[pl.ds(base + t * blk, blk)], semw)
    for d in w_d:
        if d is not None:
            d.wait()


def _sc_gather(atoms2, idx1, idx2):
    n, d = atoms2.shape
    e = idx1.shape[0]
    ew = e // _NW
    assert ew * _NW == e
    blk = 1000
    nblk = ew // blk
    assert nblk * blk == ew
    mesh = plsc.VectorSubcoreMesh(core_axis_name="c", subcore_axis_name="s")
    stage = max(blk, (n // _NS) // 8 * 8 + (n - (n // _NS) // 8 * 8 * _NS))
    f = pl.kernel(
        functools.partial(_gather_body, n, nblk, blk),
        out_type=(
            jax.ShapeDtypeStruct((e, d), jnp.float32),
            jax.ShapeDtypeStruct((e, d), jnp.float32),
        ),
        mesh=mesh,
        scratch_types=[
            pltpu.VMEM((blk,), jnp.int32),
            pltpu.VMEM((blk,), jnp.int32),
            pltpu.VMEM((stage, d), jnp.float32),
            pltpu.VMEM((stage, d), jnp.float32),
            pltpu.VMEM_SHARED((n, d), jnp.float32),
            pltpu.SemaphoreType.DMA,
            pltpu.SemaphoreType.DMA,
            pltpu.SemaphoreType.DMA,
        ],
        compiler_params=pltpu.CompilerParams(use_tc_tiling_on_sc=False),
    )
    return f(atoms2, idx1, idx2)


def _scatter_body(n, nblk, blk, zeros32_hbm, zeros16_hbm, ones_hbm,
                  idx_hbm, vals_hbm, sums_out, counts_out,
                  idx_v0, idx_v1, vals_v0, vals_v1, ones_v, cz_v,
                  sums_sp, counts_sp, semi, semv, sems):
    c = lax.axis_index("c")
    s = lax.axis_index("s")
    chunk = (n // _NS) // 8 * 8
    rem = n - chunk * _NS
    ew = nblk * blk
    base = (c * _NS + s) * ew

    pltpu.sync_copy(ones_hbm, ones_v)

    def zero_slice(off, ln):
        pltpu.sync_copy(zeros32_hbm.at[pl.ds(off, ln)],
                        vals_v0.at[pl.ds(0, ln)])
        pltpu.sync_copy(vals_v0.at[pl.ds(0, ln)], sums_sp.at[pl.ds(off, ln)])
        pltpu.sync_copy(zeros16_hbm.at[pl.ds(off, ln)],
                        cz_v.at[pl.ds(0, ln)])
        pltpu.sync_copy(cz_v.at[pl.ds(0, ln)],
                        counts_sp.at[pl.ds(off, ln)])

    zero_slice(s * chunk, chunk)
    if rem:
        @pl.when(s == 0)
        def _():
            zero_slice(chunk * _NS, rem)
    plsc.subcore_barrier()

    idx_bufs = (idx_v0, idx_v1)
    val_bufs = (vals_v0, vals_v1)
    s_d = [None, None]

    i_d = [(pltpu.async_copy(idx_hbm.at[pl.ds(base, blk)], idx_v0, semi),
            pltpu.async_copy(vals_hbm.at[pl.ds(base, blk)], vals_v0, semv)),
           None]
    for t in range(nblk):
        b = t % 2
        nb = (t + 1) % 2
        if t + 1 < nblk:
            if s_d[nb] is not None:
                for dsc in s_d[nb]:
                    dsc.wait()
                s_d[nb] = None
            off = base + (t + 1) * blk
            i_d[nb] = (
                pltpu.async_copy(idx_hbm.at[pl.ds(off, blk)],
                                 idx_bufs[nb], semi),
                pltpu.async_copy(vals_hbm.at[pl.ds(off, blk)],
                                 val_bufs[nb], semv))
        i_d[b][0].wait()
        i_d[b][1].wait()
        subs = []
        off_s = 0
        while off_s < blk:
            ln = min(128, blk - off_s)
            subs.append(pltpu.async_copy(
                val_bufs[b].at[pl.ds(off_s, ln)],
                sums_sp.at[idx_bufs[b].at[pl.ds(off_s, ln)]], sems,
                add=True))
            subs.append(pltpu.async_copy(
                ones_v.at[pl.ds(0, ln)],
                counts_sp.at[idx_bufs[b].at[pl.ds(off_s, ln)]], sems,
                add=True))
            off_s += ln
        s_d[b] = subs
    for dd in s_d:
        if dd is not None:
            for dsc in dd:
                dsc.wait()

    plsc.subcore_barrier()

    def out_slice(off, ln):
        pltpu.sync_copy(sums_sp.at[pl.ds(off, ln)], vals_v0.at[pl.ds(0, ln)])
        pltpu.sync_copy(vals_v0.at[pl.ds(0, ln)],
                        sums_out.at[c].at[pl.ds(off, ln)])
        pltpu.sync_copy(counts_sp.at[pl.ds(off, ln)],
                        cz_v.at[pl.ds(0, ln)])
        pltpu.sync_copy(cz_v.at[pl.ds(0, ln)],
                        counts_out.at[c].at[pl.ds(off, ln)])

    out_slice(s * chunk, chunk)
    if rem:
        @pl.when(s == 0)
        def _():
            out_slice(chunk * _NS, rem)


def _sc_scatter(n, idx1, vals):
    e = vals.shape[0]
    d = vals.shape[1]
    ew = e // _NW
    assert ew * _NW == e
    blk = 1000
    nblk = ew // blk
    assert nblk * blk == ew
    chunk = (n // _NS) // 8 * 8
    rem = n - chunk * _NS
    stage = max(blk, chunk + rem)
    mesh = plsc.VectorSubcoreMesh(core_axis_name="c", subcore_axis_name="s")
    f = pl.kernel(
        functools.partial(_scatter_body, n, nblk, blk),
        out_type=(
            jax.ShapeDtypeStruct((_NC, n, d), jnp.float32),
            jax.ShapeDtypeStruct((_NC, n, 16), jnp.float32),
        ),
        mesh=mesh,
        scratch_types=[
            pltpu.VMEM((blk,), jnp.int32),
            pltpu.VMEM((blk,), jnp.int32),
            pltpu.VMEM((stage, d), jnp.float32),
            pltpu.VMEM((stage, d), jnp.float32),
            pltpu.VMEM((blk, 16), jnp.float32),
            pltpu.VMEM((stage, 16), jnp.float32),
            pltpu.VMEM_SHARED((n, d), jnp.float32),
            pltpu.VMEM_SHARED((n, 16), jnp.float32),
            pltpu.SemaphoreType.DMA,
            pltpu.SemaphoreType.DMA,
            pltpu.SemaphoreType.DMA,
        ],
        compiler_params=pltpu.CompilerParams(use_tc_tiling_on_sc=False),
    )
    zeros32 = jnp.zeros((n, d), jnp.float32)
    zeros16 = jnp.zeros((n, 16), jnp.float32)
    ones = jnp.ones((blk, 16), jnp.float32)
    return f(zeros32, zeros16, ones, idx1, vals)


def _edge_body(g1_ref, g2_ref, b_ref, st_ref,
               w1a_ref, w1b_ref, w1c_ref, w1d_ref, b1_ref,
               w2_ref, b2_ref, w3_ref, b3_ref,
               out_ref, bsum_ref):
    i = pl.program_id(0)
    bf = jnp.bfloat16
    stt = jnp.dot(st_ref[...], w1d_ref[...],
                  preferred_element_type=jnp.float32) + b1_ref[...]
    stt4 = jnp.concatenate([stt, stt, stt, stt], axis=1)
    x = (jnp.dot(g1_ref[...].astype(bf), w1a_ref[...],
                 preferred_element_type=jnp.float32)
         + jnp.dot(g2_ref[...].astype(bf), w1b_ref[...],
                   preferred_element_type=jnp.float32)
         + jnp.dot(b_ref[...].astype(bf), w1c_ref[...],
                   preferred_element_type=jnp.float32)
         + stt4)
    h = _softplus(x)
    h = _softplus(jnp.dot(h.astype(bf), w2_ref[...],
                          preferred_element_type=jnp.float32) + b2_ref[...])
    o = _softplus(jnp.dot(h.astype(bf), w3_ref[...],
                          preferred_element_type=jnp.float32) + b3_ref[...])
    out_ref[...] = o

    @pl.when(i == 0)
    def _():
        bsum_ref[...] = jnp.zeros_like(bsum_ref)

    bsum_ref[0:1, :] += jnp.sum(o, axis=0, keepdims=True)


def _tc_edge_mlp(g1p, g2p, bondsp, st_row, w1a, w1b, w1c, w1d,
                 b1, w2, b2, w3, b3):
    e4 = g1p.shape[0]
    bf = jnp.bfloat16
    w1a_bd = _bd4(w1a).astype(bf)
    w1b_bd = _bd4(w1b).astype(bf)
    w1c_bd = _bd4(w1c).astype(bf)
    b2t = jnp.tile(b2, 4)[None, :]
    w2_bd = _bd4(w2).astype(bf)
    w3_bd = _bd4(w3).astype(bf)
    b3t = jnp.tile(b3, 4)[None, :]
    r = 800
    grid = e4 // r
    assert grid * r == e4
    row_spec = pl.BlockSpec((r, 128), lambda i: (i, 0))

    def fs(x):
        return pl.BlockSpec(x.shape, lambda i: tuple(0 for _ in x.shape))

    args = (g1p, g2p, bondsp, st_row, w1a_bd, w1b_bd, w1c_bd,
            w1d, b1[None, :], w2_bd, b2t, w3_bd, b3t)
    out, bsum = pl.pallas_call(
        _edge_body,
        grid=(grid,),
        in_specs=[row_spec, row_spec, row_spec] + [fs(a) for a in args[3:]],
        out_specs=[pl.BlockSpec((r, 128), lambda i: (i, 0)),
                   pl.BlockSpec((8, 128), lambda i: (0, 0))],
        out_shape=[jax.ShapeDtypeStruct((e4, 128), jnp.float32),
                   jax.ShapeDtypeStruct((8, 128), jnp.float32)],
    )(*args)
    return out, bsum


def _node_body(e_edges, n_atoms, sums_ref, counts_ref,
               atoms_ref, st_ref, bsum_ref, fold_ref,
               wv1a_ref, wv1b_ref, wv1c_ref, bv1_ref, wv2_ref, bv2_ref,
               wv3_ref, bv3_ref,
               wu1a_ref, wu1b_ref, wu1c_ref, bu1_ref, wu2_ref, bu2_ref,
               wu3_ref, bu3_ref,
               atoms_out_ref, state_out_ref):
    bf = jnp.bfloat16
    ssum = sums_ref[0] + sums_ref[1]
    cnt = counts_ref[...]
    bta = ssum / cnt
    st = st_ref[...]
    stt = (jnp.dot(st, wv1c_ref[...], preferred_element_type=jnp.float32)
           + bv1_ref[...])
    stt4 = jnp.concatenate([stt, stt, stt, stt], axis=1)
    x = (jnp.dot(bta.astype(bf), wv1a_ref[...],
                 preferred_element_type=jnp.float32)
         + jnp.dot(atoms_ref[...].astype(bf), wv1b_ref[...],
                   preferred_element_type=jnp.float32)
         + stt4)
    h = _softplus(x)
    h = _softplus(jnp.dot(h.astype(bf), wv2_ref[...],
                          preferred_element_type=jnp.float32) + bv2_ref[...])
    atoms_new = _softplus(
        jnp.dot(h.astype(bf), wv3_ref[...],
                preferred_element_type=jnp.float32) + bv3_ref[...])
    atoms_out_ref[...] = atoms_new

    fold = fold_ref[...]
    asum = jnp.dot(jnp.sum(atoms_new, axis=0, keepdims=True), fold,
                   preferred_element_type=jnp.float32)
    bsum = jnp.dot(jnp.sum(bsum_ref[...], axis=0, keepdims=True),
                   fold, preferred_element_type=jnp.float32)
    bts = bsum * (1.0 / e_edges)
    ats = asum * (1.0 / n_atoms)
    xs = (jnp.dot(bts, wu1a_ref[...], preferred_element_type=jnp.float32)
          + jnp.dot(ats, wu1b_ref[...], preferred_element_type=jnp.float32)
          + jnp.dot(st, wu1c_ref[...], preferred_element_type=jnp.float32)
          + bu1_ref[...])
    hs = _softplus(xs)
    hs = _softplus(jnp.dot(hs, wu2_ref[...],
                           preferred_element_type=jnp.float32) + bu2_ref[...])
    sn = _softplus(jnp.dot(hs, wu3_ref[...],
                           preferred_element_type=jnp.float32) + bu3_ref[...])
    state_out_ref[...] = jnp.broadcast_to(sn, state_out_ref.shape)


def _tc_node_state(e_edges, n_atoms, sums_p, counts_p, atoms_p,
                   st_row, bsum, params):
    n4 = atoms_p.shape[0]
    fold = jnp.tile(jnp.eye(32, dtype=jnp.float32), (4, 1))

    def fs(x):
        return pl.BlockSpec(x.shape, lambda: tuple(0 for _ in x.shape))

    args = (sums_p, counts_p, atoms_p, st_row, bsum, fold) + tuple(params)
    atoms_new, state_new = pl.pallas_call(
        functools.partial(_node_body, e_edges, n_atoms),
        grid=(),
        in_specs=[fs(a) for a in args],
        out_specs=[pl.BlockSpec((n4, 128), lambda: (0, 0)),
                   pl.BlockSpec((8, 32), lambda: (0, 0))],
        out_shape=[jax.ShapeDtypeStruct((n4, 128), jnp.float32),
                   jax.ShapeDtypeStruct((8, 32), jnp.float32)],
    )(*args)
    return atoms_new, state_new


def kernel(bonds, bond_atom_1, bond_atom_2, atoms, state,
           W_e1, b_e1, W_e2, b_e2, W_e3, b_e3,
           W_v1, b_v1, W_v2, b_v2, W_v3, b_v3,
           W_u1, b_u1, W_u2, b_u2, W_u3, b_u3):
    b, e, d = bonds.shape
    n = atoms.shape[1]
    e4, n4 = e // 4, n // 4
    bonds2 = bonds.reshape(e, d)
    atoms2 = atoms.reshape(n, d)
    idx1 = bond_atom_1.reshape(e).astype(jnp.int32)
    idx2 = bond_atom_2.reshape(e).astype(jnp.int32)
    st_row = state.reshape(1, d)

    g1, g2 = _sc_gather(atoms2, idx1, idx2)
    bn_p, bsum = _tc_edge_mlp(
        g1.reshape(e4, 128), g2.reshape(e4, 128), bonds2.reshape(e4, 128),
        st_row,
        W_e1[0:32], W_e1[32:64], W_e1[64:96], W_e1[96:128], b_e1,
        W_e2, b_e2, W_e3, b_e3)

    sums, counts = _sc_scatter(n, idx1, bn_p.reshape(e, 32))

    counts16 = counts[0] + counts[1]
    counts_p = jnp.tile(counts16, (1, 2)).reshape(n4, 128)

    bf = jnp.bfloat16
    node_params = (
        _bd4(W_v1[0:32]).astype(bf), _bd4(W_v1[32:64]).astype(bf),
        W_v1[64:96], b_v1.reshape(1, -1),
        _bd4(W_v2).astype(bf), jnp.tile(b_v2, 4)[None, :],
        _bd4(W_v3).astype(bf), jnp.tile(b_v3, 4)[None, :],
        W_u1[0:32], W_u1[32:64], W_u1[64:96], b_u1.reshape(1, -1),
        W_u2, b_u2.reshape(1, -1), W_u3, b_u3.reshape(1, -1))
    atoms_new, state_new = _tc_node_state(
        float(e), float(n), sums.reshape(2, n4, 128), counts_p,
        atoms2.reshape(n4, 128), st_row, bsum, node_params)

    return (bn_p.reshape(b, e, 32),
            atoms_new.reshape(b, n, 32),
            state_new[0:1, :].reshape(b, 1, 32))


# speedup vs baseline: 1.1684x; 1.0190x over previous
import functools

import jax
import jax.numpy as jnp
from jax import lax
from jax.experimental import pallas as pl
from jax.experimental.pallas import tpu as pltpu
from jax.experimental.pallas import tpu_sc as plsc

_NC = 2
_NS = 16
_NW = _NC * _NS


def _softplus(x):
    return jnp.maximum(x, 0.0) + jnp.log1p(jnp.exp(-jnp.abs(x)))


def _bd4(w):
    a, b = w.shape
    z = jnp.zeros((a, b), w.dtype)
    rows = [jnp.concatenate([w if i == j else z for j in range(4)], axis=1)
            for i in range(4)]
    return jnp.concatenate(rows, axis=0)


def _gather_body(n, nblk, blk, atoms_hbm, idx1_hbm, idx2_hbm,
                 out1_hbm, out2_hbm, idx_v0, idx_v1, rows_v0, rows_v1,
                 atoms_sp, semi, semg, semw):
    c = lax.axis_index("c")
    s = lax.axis_index("s")
    wid = s * _NC + c
    ew = nblk * blk
    base = wid * ew

    chunk = (n // _NS) // 8 * 8
    rem = n - chunk * _NS
    pltpu.sync_copy(atoms_hbm.at[pl.ds(s * chunk, chunk)],
                    rows_v0.at[pl.ds(0, chunk)])
    pltpu.sync_copy(rows_v0.at[pl.ds(0, chunk)],
                    atoms_sp.at[pl.ds(s * chunk, chunk)])
    if rem:
        @pl.when(s == 0)
        def _():
            pltpu.sync_copy(atoms_hbm.at[pl.ds(chunk * _NS, rem)],
                            rows_v0.at[pl.ds(0, rem)])
            pltpu.sync_copy(rows_v0.at[pl.ds(0, rem)],
                            atoms_sp.at[pl.ds(chunk * _NS, rem)])
    plsc.subcore_barrier()

    idx_bufs = (idx_v0, idx_v1)
    row_bufs = (rows_v0, rows_v1)
    tasks = ([(idx1_hbm, out1_hbm, t) for t in range(nblk)]
             + [(idx2_hbm, out2_hbm, t) for t in range(nblk)])
    nt = len(tasks)
    w_d = [None, None]

    ih0, _, t0 = tasks[0]
    i_d = [pltpu.async_copy(ih0.at[pl.ds(base + t0 * blk, blk)],
                            idx_bufs[0], semi), None]
    for i, (ih, oh, t) in enumerate(tasks):
        b = i % 2
        nb = (i + 1) % 2
        if i + 1 < nt:
            ihn, _, tn = tasks[i + 1]
            i_d[nb] = pltpu.async_copy(
                ihn.at[pl.ds(base + tn * blk, blk)], idx_bufs[nb], semi)
        i_d[b].wait()
        if w_d[b] is not None:
            w_d[b].wait()
        pltpu.async_copy(atoms_sp.at[idx_bufs[b]], row_bufs[b], semg).wait()
        w_d[b] = pltpu.async_copy(row_bufs[b],
                                  oh.at[pl.ds(base + t * blk, blk)], semw)
    for d in w_d:
        if d is not None:
            d.wait()


def _sc_gather(atoms2, idx1, idx2):
    n, d = atoms2.shape
    e = idx1.shape[0]
    ew = e // _NW
    assert ew * _NW == e
    blk = 1000
    nblk = ew // blk
    assert nblk * blk == ew
    mesh = plsc.VectorSubcoreMesh(core_axis_name="c", subcore_axis_name="s")
    stage = max(blk, (n // _NS) // 8 * 8 + (n - (n // _NS) // 8 * 8 * _NS))
    f = pl.kernel(
        functools.partial(_gather_body, n, nblk, blk),
        out_type=(
            jax.ShapeDtypeStruct((e, d), jnp.float32),
            jax.ShapeDtypeStruct((e, d), jnp.float32),
        ),
        mesh=mesh,
        scratch_types=[
            pltpu.VMEM((blk,), jnp.int32),
            pltpu.VMEM((blk,), jnp.int32),
            pltpu.VMEM((stage, d), jnp.float32),
            pltpu.VMEM((stage, d), jnp.float32),
            pltpu.VMEM_SHARED((n, d), jnp.float32),
            pltpu.SemaphoreType.DMA,
            pltpu.SemaphoreType.DMA,
            pltpu.SemaphoreType.DMA,
        ],
        compiler_params=pltpu.CompilerParams(use_tc_tiling_on_sc=False),
    )
    return f(atoms2, idx1, idx2)


def _scatter_body(n, nblk, blk, zeros32_hbm, zeros16_hbm, ones_hbm,
                  idx_hbm, vals_hbm, sums_out, counts_out,
                  idx_v0, idx_v1, vals_v0, vals_v1, ones_v, cz_v,
                  sums_sp, counts_sp, semi, semv, sems):
    c = lax.axis_index("c")
    s = lax.axis_index("s")
    chunk = (n // _NS) // 8 * 8
    rem = n - chunk * _NS
    ew = nblk * blk
    base = (c * _NS + s) * ew

    pltpu.sync_copy(ones_hbm, ones_v)

    def zero_slice(off, ln):
        pltpu.sync_copy(zeros32_hbm.at[pl.ds(off, ln)],
                        vals_v0.at[pl.ds(0, ln)])
        pltpu.sync_copy(vals_v0.at[pl.ds(0, ln)], sums_sp.at[pl.ds(off, ln)])
        pltpu.sync_copy(zeros16_hbm.at[pl.ds(off, ln)],
                        cz_v.at[pl.ds(0, ln)])
        pltpu.sync_copy(cz_v.at[pl.ds(0, ln)],
                        counts_sp.at[pl.ds(off, ln)])

    zero_slice(s * chunk, chunk)
    if rem:
        @pl.when(s == 0)
        def _():
            zero_slice(chunk * _NS, rem)
    plsc.subcore_barrier()

    idx_bufs = (idx_v0, idx_v1)
    val_bufs = (vals_v0, vals_v1)
    s_d = [None, None]

    i_d = [(pltpu.async_copy(idx_hbm.at[pl.ds(base, blk)], idx_v0, semi),
            pltpu.async_copy(vals_hbm.at[pl.ds(base, blk)], vals_v0, semv)),
           None]
    for t in range(nblk):
        b = t % 2
        nb = (t + 1) % 2
        if t + 1 < nblk:
            if s_d[nb] is not None:
                for dsc in s_d[nb]:
                    dsc.wait()
                s_d[nb] = None
            off = base + (t + 1) * blk
            i_d[nb] = (
                pltpu.async_copy(idx_hbm.at[pl.ds(off, blk)],
                                 idx_bufs[nb], semi),
                pltpu.async_copy(vals_hbm.at[pl.ds(off, blk)],
                                 val_bufs[nb], semv))
        i_d[b][0].wait()
        i_d[b][1].wait()
        subs = []
        off_s = 0
        while off_s < blk:
            ln = min(128, blk - off_s)
            subs.append(pltpu.async_copy(
                val_bufs[b].at[pl.ds(off_s, ln)],
                sums_sp.at[idx_bufs[b].at[pl.ds(off_s, ln)]], sems,
                add=True))
            subs.append(pltpu.async_copy(
                ones_v.at[pl.ds(0, ln)],
                counts_sp.at[idx_bufs[b].at[pl.ds(off_s, ln)]], sems,
                add=True))
            off_s += ln
        s_d[b] = subs
    for dd in s_d:
        if dd is not None:
            for dsc in dd:
                dsc.wait()

    plsc.subcore_barrier()

    def out_slice(off, ln):
        pltpu.sync_copy(sums_sp.at[pl.ds(off, ln)], vals_v0.at[pl.ds(0, ln)])
        pltpu.sync_copy(vals_v0.at[pl.ds(0, ln)],
                        sums_out.at[c].at[pl.ds(off, ln)])
        pltpu.sync_copy(counts_sp.at[pl.ds(off, ln)],
                        cz_v.at[pl.ds(0, ln)])
        pltpu.sync_copy(cz_v.at[pl.ds(0, ln)],
                        counts_out.at[c].at[pl.ds(off, ln)])

    out_slice(s * chunk, chunk)
    if rem:
        @pl.when(s == 0)
        def _():
            out_slice(chunk * _NS, rem)


def _sc_scatter(n, idx1, vals):
    e = vals.shape[0]
    d = vals.shape[1]
    ew = e // _NW
    assert ew * _NW == e
    blk = 1000
    nblk = ew // blk
    assert nblk * blk == ew
    chunk = (n // _NS) // 8 * 8
    rem = n - chunk * _NS
    stage = max(blk, chunk + rem)
    mesh = plsc.VectorSubcoreMesh(core_axis_name="c", subcore_axis_name="s")
    f = pl.kernel(
        functools.partial(_scatter_body, n, nblk, blk),
        out_type=(
            jax.ShapeDtypeStruct((_NC, n, d), jnp.float32),
            jax.ShapeDtypeStruct((_NC, n, 16), jnp.float32),
        ),
        mesh=mesh,
        scratch_types=[
            pltpu.VMEM((blk,), jnp.int32),
            pltpu.VMEM((blk,), jnp.int32),
            pltpu.VMEM((stage, d), jnp.float32),
            pltpu.VMEM((stage, d), jnp.float32),
            pltpu.VMEM((blk, 16), jnp.float32),
            pltpu.VMEM((stage, 16), jnp.float32),
            pltpu.VMEM_SHARED((n, d), jnp.float32),
            pltpu.VMEM_SHARED((n, 16), jnp.float32),
            pltpu.SemaphoreType.DMA,
            pltpu.SemaphoreType.DMA,
            pltpu.SemaphoreType.DMA,
        ],
        compiler_params=pltpu.CompilerParams(use_tc_tiling_on_sc=False),
    )
    zeros32 = jnp.zeros((n, d), jnp.float32)
    zeros16 = jnp.zeros((n, 16), jnp.float32)
    ones = jnp.ones((blk, 16), jnp.float32)
    return f(zeros32, zeros16, ones, idx1, vals)


def _edge_body(g1_ref, g2_ref, b_ref, st_ref,
               w1a_ref, w1b_ref, w1c_ref, w1d_ref, b1_ref,
               w2_ref, b2_ref, w3_ref, b3_ref,
               out_ref, bsum_ref):
    i = pl.program_id(0)
    bf = jnp.bfloat16
    stt = jnp.dot(st_ref[...], w1d_ref[...],
                  preferred_element_type=jnp.float32) + b1_ref[...]
    stt4 = jnp.concatenate([stt, stt, stt, stt], axis=1)
    x = (jnp.dot(g1_ref[...].astype(bf), w1a_ref[...],
                 preferred_element_type=jnp.float32)
         + jnp.dot(g2_ref[...].astype(bf), w1b_ref[...],
                   preferred_element_type=jnp.float32)
         + jnp.dot(b_ref[...].astype(bf), w1c_ref[...],
                   preferred_element_type=jnp.float32)
         + stt4)
    h = _softplus(x)
    h = _softplus(jnp.dot(h.astype(bf), w2_ref[...],
                          preferred_element_type=jnp.float32) + b2_ref[...])
    o = _softplus(jnp.dot(h.astype(bf), w3_ref[...],
                          preferred_element_type=jnp.float32) + b3_ref[...])
    out_ref[...] = o

    @pl.when(i == 0)
    def _():
        bsum_ref[...] = jnp.zeros_like(bsum_ref)

    bsum_ref[0:1, :] += jnp.sum(o, axis=0, keepdims=True)


def _tc_edge_mlp(g1p, g2p, bondsp, st_row, w1a, w1b, w1c, w1d,
                 b1, w2, b2, w3, b3):
    e4 = g1p.shape[0]
    bf = jnp.bfloat16
    w1a_bd = _bd4(w1a).astype(bf)
    w1b_bd = _bd4(w1b).astype(bf)
    w1c_bd = _bd4(w1c).astype(bf)
    b2t = jnp.tile(b2, 4)[None, :]
    w2_bd = _bd4(w2).astype(bf)
    w3_bd = _bd4(w3).astype(bf)
    b3t = jnp.tile(b3, 4)[None, :]
    r = 1600
    grid = e4 // r
    assert grid * r == e4
    row_spec = pl.BlockSpec((r, 128), lambda i: (i, 0))

    def fs(x):
        return pl.BlockSpec(x.shape, lambda i: tuple(0 for _ in x.shape))

    args = (g1p, g2p, bondsp, st_row, w1a_bd, w1b_bd, w1c_bd,
            w1d, b1[None, :], w2_bd, b2t, w3_bd, b3t)
    out, bsum = pl.pallas_call(
        _edge_body,
        grid=(grid,),
        in_specs=[row_spec, row_spec, row_spec] + [fs(a) for a in args[3:]],
        out_specs=[pl.BlockSpec((r, 128), lambda i: (i, 0)),
                   pl.BlockSpec((8, 128), lambda i: (0, 0))],
        out_shape=[jax.ShapeDtypeStruct((e4, 128), jnp.float32),
                   jax.ShapeDtypeStruct((8, 128), jnp.float32)],
    )(*args)
    return out, bsum


def _node_body(e_edges, n_atoms, sums_ref, counts_ref,
               atoms_ref, st_ref, bsum_ref, fold_ref,
               wv1a_ref, wv1b_ref, wv1c_ref, bv1_ref, wv2_ref, bv2_ref,
               wv3_ref, bv3_ref,
               wu1a_ref, wu1b_ref, wu1c_ref, bu1_ref, wu2_ref, bu2_ref,
               wu3_ref, bu3_ref,
               atoms_out_ref, state_out_ref):
    bf = jnp.bfloat16
    ssum = sums_ref[0] + sums_ref[1]
    cnt = counts_ref[...]
    bta = ssum / cnt
    st = st_ref[...]
    stt = (jnp.dot(st, wv1c_ref[...], preferred_element_type=jnp.float32)
           + bv1_ref[...])
    stt4 = jnp.concatenate([stt, stt, stt, stt], axis=1)
    x = (jnp.dot(bta.astype(bf), wv1a_ref[...],
                 preferred_element_type=jnp.float32)
         + jnp.dot(atoms_ref[...].astype(bf), wv1b_ref[...],
                   preferred_element_type=jnp.float32)
         + stt4)
    h = _softplus(x)
    h = _softplus(jnp.dot(h.astype(bf), wv2_ref[...],
                          preferred_element_type=jnp.float32) + bv2_ref[...])
    atoms_new = _softplus(
        jnp.dot(h.astype(bf), wv3_ref[...],
                preferred_element_type=jnp.float32) + bv3_ref[...])
    atoms_out_ref[...] = atoms_new

    fold = fold_ref[...]
    asum = jnp.dot(jnp.sum(atoms_new, axis=0, keepdims=True), fold,
                   preferred_element_type=jnp.float32)
    bsum = jnp.dot(jnp.sum(bsum_ref[...], axis=0, keepdims=True),
                   fold, preferred_element_type=jnp.float32)
    bts = bsum * (1.0 / e_edges)
    ats = asum * (1.0 / n_atoms)
    xs = (jnp.dot(bts, wu1a_ref[...], preferred_element_type=jnp.float32)
          + jnp.dot(ats, wu1b_ref[...], preferred_element_type=jnp.float32)
          + jnp.dot(st, wu1c_ref[...], preferred_element_type=jnp.float32)
          + bu1_ref[...])
    hs = _softplus(xs)
    hs = _softplus(jnp.dot(hs, wu2_ref[...],
                           preferred_element_type=jnp.float32) + bu2_ref[...])
    sn = _softplus(jnp.dot(hs, wu3_ref[...],
                           preferred_element_type=jnp.float32) + bu3_ref[...])
    state_out_ref[...] = jnp.broadcast_to(sn, state_out_ref.shape)


def _tc_node_state(e_edges, n_atoms, sums_p, counts_p, atoms_p,
                   st_row, bsum, params):
    n4 = atoms_p.shape[0]
    fold = jnp.tile(jnp.eye(32, dtype=jnp.float32), (4, 1))

    def fs(x):
        return pl.BlockSpec(x.shape, lambda: tuple(0 for _ in x.shape))

    args = (sums_p, counts_p, atoms_p, st_row, bsum, fold) + tuple(params)
    atoms_new, state_new = pl.pallas_call(
        functools.partial(_node_body, e_edges, n_atoms),
        grid=(),
        in_specs=[fs(a) for a in args],
        out_specs=[pl.BlockSpec((n4, 128), lambda: (0, 0)),
                   pl.BlockSpec((8, 32), lambda: (0, 0))],
        out_shape=[jax.ShapeDtypeStruct((n4, 128), jnp.float32),
                   jax.ShapeDtypeStruct((8, 32), jnp.float32)],
    )(*args)
    return atoms_new, state_new


def kernel(bonds, bond_atom_1, bond_atom_2, atoms, state,
           W_e1, b_e1, W_e2, b_e2, W_e3, b_e3,
           W_v1, b_v1, W_v2, b_v2, W_v3, b_v3,
           W_u1, b_u1, W_u2, b_u2, W_u3, b_u3):
    b, e, d = bonds.shape
    n = atoms.shape[1]
    e4, n4 = e // 4, n // 4
    bonds2 = bonds.reshape(e, d)
    atoms2 = atoms.reshape(n, d)
    idx1 = bond_atom_1.reshape(e).astype(jnp.int32)
    idx2 = bond_atom_2.reshape(e).astype(jnp.int32)
    st_row = state.reshape(1, d)

    g1, g2 = _sc_gather(atoms2, idx1, idx2)
    bn_p, bsum = _tc_edge_mlp(
        g1.reshape(e4, 128), g2.reshape(e4, 128), bonds2.reshape(e4, 128),
        st_row,
        W_e1[0:32], W_e1[32:64], W_e1[64:96], W_e1[96:128], b_e1,
        W_e2, b_e2, W_e3, b_e3)

    sums, counts = _sc_scatter(n, idx1, bn_p.reshape(e, 32))

    counts16 = counts[0] + counts[1]
    counts_p = jnp.tile(counts16, (1, 2)).reshape(n4, 128)

    bf = jnp.bfloat16
    node_params = (
        _bd4(W_v1[0:32]).astype(bf), _bd4(W_v1[32:64]).astype(bf),
        W_v1[64:96], b_v1.reshape(1, -1),
        _bd4(W_v2).astype(bf), jnp.tile(b_v2, 4)[None, :],
        _bd4(W_v3).astype(bf), jnp.tile(b_v3, 4)[None, :],
        W_u1[0:32], W_u1[32:64], W_u1[64:96], b_u1.reshape(1, -1),
        W_u2, b_u2.reshape(1, -1), W_u3, b_u3.reshape(1, -1))
    atoms_new, state_new = _tc_node_state(
        float(e), float(n), sums.reshape(2, n4, 128), counts_p,
        atoms2.reshape(n4, 128), st_row, bsum, node_params)

    return (bn_p.reshape(b, e, 32),
            atoms_new.reshape(b, n, 32),
            state_new[0:1, :].reshape(b, 1, 32))
